# baseline (speedup 1.0000x reference)
import functools
from math import log

import numpy as np
import jax
import jax.numpy as jnp
from jax import lax
from jax.experimental import pallas as pl
from jax.experimental.pallas import tpu as pltpu
from jax.experimental.pallas import tpu_sc as plsc

_ALPHA = 0.1
_BETA = log(0.5 / 2 + 1.0)

_NC = 2
_NS = 16
_L = 16
_NW = _NC * _NS


def _sc_gather_sum(x2d, idx1d, B, N, K, C):
    total = B * N
    CH = 8
    assert N % CH == 0
    assert K == _L
    npw = ((total + _NW - 1) // _NW + CH - 1) // CH * CH
    ncw = npw // CH
    dg = C // _L
    NB = 6

    ncw_pad = (ncw + 7) // 8 * 8
    pad = _NW * npw * K - total * K
    idxp = jnp.concatenate([idx1d, jnp.zeros((pad,), jnp.int32)])
    idx3d = idxp.reshape(_NW, ncw, CH * K)
    idx3d = jnp.pad(idx3d, ((0, 0), (0, ncw_pad - ncw), (0, 0)))
    idx2d = idx3d.reshape(_NW * ncw_pad, CH * K)

    mesh = plsc.VectorSubcoreMesh(core_axis_name="c", subcore_axis_name="s")

    @functools.partial(
        pl.kernel,
        mesh=mesh,
        out_type=jax.ShapeDtypeStruct((total, C), jnp.float32),
        scratch_types=[
            pltpu.VMEM((ncw_pad, CH * K), jnp.int32),
            pltpu.VMEM((NB, CH * K, C), jnp.float32),
            pltpu.VMEM((CH, C), jnp.float32),
            pltpu.SemaphoreType.DMA,
            pltpu.SemaphoreType.DMA,
            pltpu.SemaphoreType.DMA,
            pltpu.SemaphoreType.DMA,
            pltpu.SemaphoreType.DMA,
            pltpu.SemaphoreType.DMA,
        ],
    )
    def k(x_hbm, idx_hbm, out_hbm, idx_v, rows_v, acc_v, *sems):
        wid = lax.axis_index("s") * _NC + lax.axis_index("c")
        base = wid * npw
        nch = jnp.maximum(jnp.minimum(total - base, npw), 0) // CH

        pltpu.sync_copy(idx_hbm.at[pl.ds(wid * ncw_pad, ncw_pad)], idx_v)

        def offs(r, carry):
            off = ((base + r * CH) // N) * N
            for q in range(CH):
                sl = pl.ds(q * _L, _L)
                idx_v[r, sl] = idx_v[r, sl] + off
            return carry

        lax.fori_loop(0, nch, offs, 0)

        def issue(c, buf):
            pltpu.async_copy(x_hbm.at[idx_v.at[c]], rows_v.at[buf], sems[buf])

        def wait(c, buf):
            pltpu.make_async_copy(
                x_hbm.at[idx_v.at[c]], rows_v.at[buf], sems[buf]
            ).wait()

        def reduce_store(c, buf):
            g0 = base + c * CH

            def node(n, carry):
                for d in range(dg):
                    sl = pl.ds(d * _L, _L)
                    acc = rows_v[buf, n * K, sl]
                    for kk in range(1, K):
                        acc = acc + rows_v[buf, n * K + kk, sl]
                    acc_v[n, sl] = acc
                return carry

            lax.fori_loop(0, CH, node, 0)
            pltpu.sync_copy(acc_v, out_hbm.at[pl.ds(g0, CH)])

        def body(i, carry):
            c0 = i * NB
            for b in range(NB):
                c = c0 + b

                @pl.when(c < nch)
                def _(c=c, b=b):
                    wait(c, b)
                    reduce_store(c, b)

                @pl.when(c + NB < nch)
                def _(c=c, b=b):
                    issue(c + NB, b)

            return carry

        for b in range(NB):
            @pl.when(b < nch)
            def _(b=b):
                issue(b, b)
        lax.fori_loop(0, (nch + NB - 1) // NB, body, 0)

    return k(x2d, idx2d)


def _tc_transpose_body(x_ref, out_ref):
    out_ref[0] = x_ref[0].T


def _tc_transpose(x3, B, N, C):
    T = 512
    grid = (B, (N + T - 1) // T)
    return pl.pallas_call(
        _tc_transpose_body,
        grid=grid,
        in_specs=[pl.BlockSpec((1, C, T), lambda b, t: (b, 0, t))],
        out_specs=pl.BlockSpec((1, T, C), lambda b, t: (b, t, 0)),
        out_shape=jax.ShapeDtypeStruct((B, N, C), jnp.float32),
    )(x3)


def _tc_combine_body(g_ref, xs_ref, x0_ref, w1_ref, w2_ref, b_ref, out_ref, *, deg):
    inv = 1.0 / deg
    c_a = (1.0 - _ALPHA) * (1.0 - _BETA)
    c_0 = _ALPHA * (1.0 - _BETA)
    aggr = (g_ref[0] + xs_ref[0]) * inv
    x0 = x0_ref[0]
    r = aggr * c_a + jnp.dot(aggr, w1_ref[...], preferred_element_type=jnp.float32) * _BETA
    r = r + x0 * c_0 + jnp.dot(x0, w2_ref[...], preferred_element_type=jnp.float32) * _BETA
    r = r + b_ref[0, 0][None, :]
    r = jnp.maximum(r, 0.0)
    out_ref[0] = r.T


def _tc_combine(gsum, xs, x0, W1, W2, bias, B, N, C, deg):
    T = 512
    grid = (B, (N + T - 1) // T)
    return pl.pallas_call(
        functools.partial(_tc_combine_body, deg=deg),
        grid=grid,
        in_specs=[
            pl.BlockSpec((1, T, C), lambda b, t: (b, t, 0)),
            pl.BlockSpec((1, T, C), lambda b, t: (b, t, 0)),
            pl.BlockSpec((1, T, C), lambda b, t: (b, t, 0)),
            pl.BlockSpec((C, C), lambda b, t: (0, 0)),
            pl.BlockSpec((C, C), lambda b, t: (0, 0)),
            pl.BlockSpec((1, 1, C), lambda b, t: (0, 0, 0)),
        ],
        out_specs=pl.BlockSpec((1, C, T), lambda b, t: (b, 0, t)),
        out_shape=jax.ShapeDtypeStruct((B, C, N), jnp.float32),
    )(gsum, xs, x0, W1, W2, bias)


def kernel(x, x_0, edge_index, W1, W2, bias):
    B, C, N, _ = x.shape
    K = edge_index.shape[-1]
    deg = K + 1

    xs = _tc_transpose(x.reshape(B, C, N), B, N, C)
    x2d = xs.reshape(B * N, C)
    idx1d = edge_index[0].reshape(B * N * K)

    gsum2d = _sc_gather_sum(x2d, idx1d, B, N, K, C)
    gsum = gsum2d.reshape(B, N, C)

    out3 = _tc_combine(gsum, xs, x_0, W1, W2, bias, B, N, C, deg)
    return out3[..., None]


# speedup vs baseline: 1.1735x; 1.1735x over previous
import functools
from math import log

import numpy as np
import jax
import jax.numpy as jnp
from jax import lax
from jax.experimental import pallas as pl
from jax.experimental.pallas import tpu as pltpu
from jax.experimental.pallas import tpu_sc as plsc

_ALPHA = 0.1
_BETA = log(0.5 / 2 + 1.0)

_NC = 2
_NS = 16
_L = 16
_NW = _NC * _NS


def _sc_gather_sum(x2d, idx1d, B, N, K, C):
    total = B * N
    CH = 8
    assert N % CH == 0
    assert K == _L
    npw = ((total + _NW - 1) // _NW + CH - 1) // CH * CH
    ncw = npw // CH
    dg = C // _L
    NB = 6

    ncw_pad = (ncw + 7) // 8 * 8
    pad = _NW * npw * K - total * K
    idxp = jnp.concatenate([idx1d, jnp.zeros((pad,), jnp.int32)])
    idx3d = idxp.reshape(_NW, ncw, CH * K)
    idx3d = jnp.pad(idx3d, ((0, 0), (0, ncw_pad - ncw), (0, 0)))
    idx2d = idx3d.reshape(_NW * ncw_pad, CH * K)

    mesh = plsc.VectorSubcoreMesh(core_axis_name="c", subcore_axis_name="s")

    @functools.partial(
        pl.kernel,
        mesh=mesh,
        out_type=jax.ShapeDtypeStruct((total, C), jnp.float32),
        scratch_types=[
            pltpu.VMEM((ncw_pad, CH * K), jnp.int32),
            pltpu.VMEM((NB, CH * K, C), jnp.float32),
            pltpu.VMEM((2, CH, C), jnp.float32),
            pltpu.SemaphoreType.DMA,
            pltpu.SemaphoreType.DMA,
            pltpu.SemaphoreType.DMA,
            pltpu.SemaphoreType.DMA,
            pltpu.SemaphoreType.DMA,
            pltpu.SemaphoreType.DMA,
            pltpu.SemaphoreType.DMA,
            pltpu.SemaphoreType.DMA,
        ],
    )
    def k(x_hbm, idx_hbm, out_hbm, idx_v, rows_v, acc_v, *sems):
        wid = lax.axis_index("s") * _NC + lax.axis_index("c")
        base = wid * npw
        nch = jnp.maximum(jnp.minimum(total - base, npw), 0) // CH

        pltpu.sync_copy(idx_hbm.at[pl.ds(wid * ncw_pad, ncw_pad)], idx_v)

        def offs(r, carry):
            off = ((base + r * CH) // N) * N
            for q in range(CH):
                sl = pl.ds(q * _L, _L)
                idx_v[r, sl] = idx_v[r, sl] + off
            return carry

        lax.fori_loop(0, nch, offs, 0)

        def issue(c, buf):
            pltpu.async_copy(x_hbm.at[idx_v.at[c]], rows_v.at[buf], sems[buf])

        def wait(c, buf):
            pltpu.make_async_copy(
                x_hbm.at[idx_v.at[c]], rows_v.at[buf], sems[buf]
            ).wait()

        def reduce_store(c, buf, p):
            g0 = base + c * CH

            @pl.when(c >= 2)
            def _():
                pltpu.make_async_copy(
                    acc_v.at[p], out_hbm.at[pl.ds(g0, CH)], sems[6 + p]
                ).wait()

            def node(n, carry):
                for d in range(dg):
                    sl = pl.ds(d * _L, _L)
                    acc = rows_v[buf, n * K, sl]
                    for kk in range(1, K):
                        acc = acc + rows_v[buf, n * K + kk, sl]
                    acc_v[p, n, sl] = acc
                return carry

            lax.fori_loop(0, CH, node, 0)
            pltpu.async_copy(acc_v.at[p], out_hbm.at[pl.ds(g0, CH)], sems[6 + p])

        def body(i, carry):
            c0 = i * NB
            for b in range(NB):
                c = c0 + b

                @pl.when(c < nch)
                def _(c=c, b=b):
                    wait(c, b)
                    reduce_store(c, b, b % 2)

                @pl.when(c + NB < nch)
                def _(c=c, b=b):
                    issue(c + NB, b)

            return carry

        for b in range(NB):
            @pl.when(b < nch)
            def _(b=b):
                issue(b, b)
        lax.fori_loop(0, (nch + NB - 1) // NB, body, 0)

        for p in range(2):
            pltpu.make_async_copy(
                acc_v.at[p], out_hbm.at[pl.ds(base, CH)], sems[6 + p]
            ).wait()

    return k(x2d, idx2d)


def _tc_combine_body(g_ref, xs_ref, x0_ref, w1_ref, w2_ref, b_ref, out_ref, *, deg):
    inv = 1.0 / deg
    c_a = (1.0 - _ALPHA) * (1.0 - _BETA)
    c_0 = _ALPHA * (1.0 - _BETA)
    aggr = (g_ref[0] + xs_ref[0]) * inv
    x0 = x0_ref[0]
    r = aggr * c_a + jnp.dot(aggr, w1_ref[...], preferred_element_type=jnp.float32) * _BETA
    r = r + x0 * c_0 + jnp.dot(x0, w2_ref[...], preferred_element_type=jnp.float32) * _BETA
    r = r + b_ref[0, 0][None, :]
    r = jnp.maximum(r, 0.0)
    out_ref[0] = r.T


def _tc_combine(gsum, xs, x0, W1, W2, bias, B, N, C, deg):
    T = 512
    grid = (B, (N + T - 1) // T)
    return pl.pallas_call(
        functools.partial(_tc_combine_body, deg=deg),
        grid=grid,
        in_specs=[
            pl.BlockSpec((1, T, C), lambda b, t: (b, t, 0)),
            pl.BlockSpec((1, T, C), lambda b, t: (b, t, 0)),
            pl.BlockSpec((1, T, C), lambda b, t: (b, t, 0)),
            pl.BlockSpec((C, C), lambda b, t: (0, 0)),
            pl.BlockSpec((C, C), lambda b, t: (0, 0)),
            pl.BlockSpec((1, 1, C), lambda b, t: (0, 0, 0)),
        ],
        out_specs=pl.BlockSpec((1, C, T), lambda b, t: (b, 0, t)),
        out_shape=jax.ShapeDtypeStruct((B, C, N), jnp.float32),
    )(gsum, xs, x0, W1, W2, bias)


def kernel(x, x_0, edge_index, W1, W2, bias):
    B, C, N, _ = x.shape
    K = edge_index.shape[-1]
    deg = K + 1

    xs = jnp.swapaxes(x[..., 0], 1, 2)
    x2d = xs.reshape(B * N, C)
    idx1d = edge_index[0].reshape(B * N * K)

    gsum2d = _sc_gather_sum(x2d, idx1d, B, N, K, C)
    gsum = gsum2d.reshape(B, N, C)

    out3 = _tc_combine(gsum, xs, x_0, W1, W2, bias, B, N, C, deg)
    return out3[..., None]


# speedup vs baseline: 1.1773x; 1.0033x over previous
import functools
from math import log

import jax
import jax.numpy as jnp
from jax import lax
from jax.experimental import pallas as pl
from jax.experimental.pallas import tpu as pltpu
from jax.experimental.pallas import tpu_sc as plsc

_ALPHA = 0.1
_BETA = log(0.5 / 2 + 1.0)

_NC = 2
_NS = 16
_L = 16
_NW = _NC * _NS


def _sc_gather_sum(x2d, idx1d, B, N, K, C):
    total = B * N
    CH = 8
    assert N % CH == 0
    assert K == _L
    npw = ((total + _NW - 1) // _NW + CH - 1) // CH * CH
    ncw = npw // CH
    dg = C // _L
    NB = 6

    ncw_pad = (ncw + 7) // 8 * 8
    pad = _NW * npw * K - total * K
    idxp = jnp.concatenate([idx1d, jnp.zeros((pad,), jnp.int32)])
    idx3d = idxp.reshape(_NW, ncw, CH * K)
    idx3d = jnp.pad(idx3d, ((0, 0), (0, ncw_pad - ncw), (0, 0)))
    idx2d = idx3d.reshape(_NW * ncw_pad, CH * K)

    mesh = plsc.VectorSubcoreMesh(core_axis_name="c", subcore_axis_name="s")

    @functools.partial(
        pl.kernel,
        mesh=mesh,
        out_type=jax.ShapeDtypeStruct((total, C), jnp.float32),
        scratch_types=[
            pltpu.VMEM((ncw_pad, CH * K), jnp.int32),
            pltpu.VMEM((NB, CH * K, C), jnp.float32),
            pltpu.VMEM((2, CH, C), jnp.float32),
            pltpu.SemaphoreType.DMA,
            pltpu.SemaphoreType.DMA,
            pltpu.SemaphoreType.DMA,
            pltpu.SemaphoreType.DMA,
            pltpu.SemaphoreType.DMA,
            pltpu.SemaphoreType.DMA,
            pltpu.SemaphoreType.DMA,
            pltpu.SemaphoreType.DMA,
        ],
    )
    def k(x_hbm, idx_hbm, out_hbm, idx_v, rows_v, acc_v, *sems):
        wid = lax.axis_index("s") * _NC + lax.axis_index("c")
        base = wid * npw
        nch = jnp.maximum(jnp.minimum(total - base, npw), 0) // CH

        pltpu.sync_copy(idx_hbm.at[pl.ds(wid * ncw_pad, ncw_pad)], idx_v)

        def offs(r, carry):
            off = ((base + r * CH) // N) * N
            for q in range(CH):
                sl = pl.ds(q * _L, _L)
                idx_v[r, sl] = idx_v[r, sl] + off
            return carry

        lax.fori_loop(0, nch, offs, 0)

        def issue(c, buf):
            pltpu.async_copy(x_hbm.at[idx_v.at[c]], rows_v.at[buf], sems[buf])

        def wait(c, buf):
            pltpu.make_async_copy(
                x_hbm.at[idx_v.at[c]], rows_v.at[buf], sems[buf]
            ).wait()

        def reduce_store(c, buf, p):
            g0 = base + c * CH

            @pl.when(c >= 2)
            def _():
                pltpu.make_async_copy(
                    acc_v.at[p], out_hbm.at[pl.ds(g0, CH)], sems[6 + p]
                ).wait()

            def node(n, carry):
                for d in range(dg):
                    sl = pl.ds(d * _L, _L)
                    acc = rows_v[buf, n * K, sl]
                    for kk in range(1, K):
                        acc = acc + rows_v[buf, n * K + kk, sl]
                    acc_v[p, n, sl] = acc
                return carry

            lax.fori_loop(0, CH, node, 0)
            pltpu.async_copy(acc_v.at[p], out_hbm.at[pl.ds(g0, CH)], sems[6 + p])

        def body(i, carry):
            c0 = i * NB
            for b in range(NB):
                c = c0 + b

                @pl.when(c < nch)
                def _(c=c, b=b):
                    wait(c, b)
                    reduce_store(c, b, b % 2)

                @pl.when(c + NB < nch)
                def _(c=c, b=b):
                    issue(c + NB, b)

            return carry

        for b in range(NB):
            @pl.when(b < nch)
            def _(b=b):
                issue(b, b)
        lax.fori_loop(0, (nch + NB - 1) // NB, body, 0)

        for p in range(2):
            pltpu.make_async_copy(
                acc_v.at[p], out_hbm.at[pl.ds(base, CH)], sems[6 + p]
            ).wait()

    return k(x2d, idx2d)


def _tc_combine_body(g_ref, xs_ref, x0_ref, w1_ref, w2_ref, b_ref, out_ref, *, deg):
    inv = 1.0 / deg
    c_a = (1.0 - _ALPHA) * (1.0 - _BETA)
    c_0 = _ALPHA * (1.0 - _BETA)
    aggr = (g_ref[0] + xs_ref[0]) * inv
    x0 = x0_ref[0]
    r = aggr * c_a + jnp.dot(aggr, w1_ref[...], preferred_element_type=jnp.float32) * _BETA
    r = r + x0 * c_0 + jnp.dot(x0, w2_ref[...], preferred_element_type=jnp.float32) * _BETA
    r = r + b_ref[0, 0][None, :]
    r = jnp.maximum(r, 0.0)
    out_ref[0] = r.T


def _tc_combine(gsum, xs, x0, W1, W2, bias, B, N, C, deg):
    T = 512
    grid = (B, (N + T - 1) // T)
    return pl.pallas_call(
        functools.partial(_tc_combine_body, deg=deg),
        grid=grid,
        in_specs=[
            pl.BlockSpec((1, T, C), lambda b, t: (b, t, 0)),
            pl.BlockSpec((1, T, C), lambda b, t: (b, t, 0)),
            pl.BlockSpec((1, T, C), lambda b, t: (b, t, 0)),
            pl.BlockSpec((C, C), lambda b, t: (0, 0)),
            pl.BlockSpec((C, C), lambda b, t: (0, 0)),
            pl.BlockSpec((1, 1, C), lambda b, t: (0, 0, 0)),
        ],
        out_specs=pl.BlockSpec((1, C, T), lambda b, t: (b, 0, t)),
        out_shape=jax.ShapeDtypeStruct((B, C, N), jnp.float32),
    )(gsum, xs, x0, W1, W2, bias)


def kernel(x, x_0, edge_index, W1, W2, bias):
    B, C, N, _ = x.shape
    K = edge_index.shape[-1]
    deg = K + 1

    xs = jnp.swapaxes(x[..., 0], 1, 2)
    x2d = xs.reshape(B * N, C)
    idx1d = edge_index[0].reshape(B * N * K)

    gsum2d = _sc_gather_sum(x2d, idx1d, B, N, K, C)
    gsum = gsum2d.reshape(B, N, C)

    out3 = _tc_combine(gsum, xs, x_0, W1, W2, bias, B, N, C, deg)
    return out3[..., None]


# speedup vs baseline: 1.2434x; 1.0562x over previous
import functools
from math import log

import jax
import jax.numpy as jnp
from jax import lax
from jax.experimental import pallas as pl
from jax.experimental.pallas import tpu as pltpu
from jax.experimental.pallas import tpu_sc as plsc

_ALPHA = 0.1
_BETA = log(0.5 / 2 + 1.0)

_NC = 2
_NS = 16
_L = 16
_NW = _NC * _NS


def _sc_gather_sum(x2d, idx1d, B, N, K, C):
    total = B * N
    CH = 8
    assert N % CH == 0
    assert K == _L
    npw = ((total + _NW - 1) // _NW + CH - 1) // CH * CH
    ncw = npw // CH
    dg = C // _L
    NB = 6

    ncw_pad = (ncw + 7) // 8 * 8
    pad = _NW * npw * K - total * K
    idxp = jnp.concatenate([idx1d, jnp.zeros((pad,), jnp.int32)])
    idx3d = idxp.reshape(_NW, ncw, CH * K)
    idx3d = jnp.pad(idx3d, ((0, 0), (0, ncw_pad - ncw), (0, 0)))
    idx2d = idx3d.reshape(_NW * ncw_pad, CH * K)

    mesh = plsc.VectorSubcoreMesh(core_axis_name="c", subcore_axis_name="s")

    @functools.partial(
        pl.kernel,
        mesh=mesh,
        out_type=jax.ShapeDtypeStruct((total, C), jnp.float32),
        scratch_types=[
            pltpu.VMEM((ncw_pad, CH * K), jnp.int32),
            pltpu.VMEM((NB, CH * K, C), jnp.float32),
            pltpu.VMEM((2, CH, C), jnp.float32),
            pltpu.SemaphoreType.DMA,
            pltpu.SemaphoreType.DMA,
            pltpu.SemaphoreType.DMA,
            pltpu.SemaphoreType.DMA,
            pltpu.SemaphoreType.DMA,
            pltpu.SemaphoreType.DMA,
            pltpu.SemaphoreType.DMA,
            pltpu.SemaphoreType.DMA,
        ],
    )
    def k(x_hbm, idx_hbm, out_hbm, idx_v, rows_v, acc_v, *sems):
        wid = lax.axis_index("s") * _NC + lax.axis_index("c")
        base = wid * npw
        nch = jnp.maximum(jnp.minimum(total - base, npw), 0) // CH

        pltpu.sync_copy(idx_hbm.at[pl.ds(wid * ncw_pad, ncw_pad)], idx_v)

        def offs(r, carry):
            off = ((base + r * CH) // N) * N
            for q in range(CH):
                sl = pl.ds(q * _L, _L)
                idx_v[r, sl] = idx_v[r, sl] + off
            return carry

        lax.fori_loop(0, nch, offs, 0)

        def issue(c, buf):
            pltpu.async_copy(x_hbm.at[idx_v.at[c]], rows_v.at[buf], sems[buf])

        def wait(c, buf):
            pltpu.make_async_copy(
                x_hbm.at[idx_v.at[c]], rows_v.at[buf], sems[buf]
            ).wait()

        def reduce_store(c, buf, p):
            g0 = base + c * CH

            @pl.when(c >= 2)
            def _():
                pltpu.make_async_copy(
                    acc_v.at[p], out_hbm.at[pl.ds(g0, CH)], sems[6 + p]
                ).wait()

            def node(n, carry):
                for d in range(dg):
                    sl = pl.ds(d * _L, _L)
                    acc = rows_v[buf, n * K, sl]
                    for kk in range(1, K):
                        acc = acc + rows_v[buf, n * K + kk, sl]
                    acc_v[p, n, sl] = acc
                return carry

            lax.fori_loop(0, CH, node, 0)
            pltpu.async_copy(acc_v.at[p], out_hbm.at[pl.ds(g0, CH)], sems[6 + p])

        def body(i, carry):
            c0 = i * NB
            for b in range(NB):
                c = c0 + b

                @pl.when(c < nch)
                def _(c=c, b=b):
                    wait(c, b)
                    reduce_store(c, b, b % 2)

                @pl.when(c + NB < nch)
                def _(c=c, b=b):
                    issue(c + NB, b)

            return carry

        for b in range(NB):
            @pl.when(b < nch)
            def _(b=b):
                issue(b, b)
        lax.fori_loop(0, (nch + NB - 1) // NB, body, 0)

        for p in range(2):
            pltpu.make_async_copy(
                acc_v.at[p], out_hbm.at[pl.ds(base, CH)], sems[6 + p]
            ).wait()

    return k(x2d, idx2d)


def _tc_combine_body(g_ref, xs_ref, x0_ref, w1_ref, w2_ref, b_ref, out_ref, *, deg):
    inv = 1.0 / deg
    c_a = (1.0 - _ALPHA) * (1.0 - _BETA)
    c_0 = _ALPHA * (1.0 - _BETA)
    aggr = (g_ref[0] + xs_ref[0]) * inv
    x0 = x0_ref[0]
    r = aggr * c_a + jnp.dot(aggr, w1_ref[...], preferred_element_type=jnp.float32) * _BETA
    r = r + x0 * c_0 + jnp.dot(x0, w2_ref[...], preferred_element_type=jnp.float32) * _BETA
    r = r + b_ref[0, 0][None, :]
    r = jnp.maximum(r, 0.0)
    out_ref[0] = r.T


def _tc_combine(gsum, xs, x0, W1, W2, bias, B, N, C, deg):
    T = 1024
    grid = (B, (N + T - 1) // T)
    return pl.pallas_call(
        functools.partial(_tc_combine_body, deg=deg),
        grid=grid,
        in_specs=[
            pl.BlockSpec((1, T, C), lambda b, t: (b, t, 0)),
            pl.BlockSpec((1, T, C), lambda b, t: (b, t, 0)),
            pl.BlockSpec((1, T, C), lambda b, t: (b, t, 0)),
            pl.BlockSpec((C, C), lambda b, t: (0, 0)),
            pl.BlockSpec((C, C), lambda b, t: (0, 0)),
            pl.BlockSpec((1, 1, C), lambda b, t: (0, 0, 0)),
        ],
        out_specs=pl.BlockSpec((1, C, T), lambda b, t: (b, 0, t)),
        out_shape=jax.ShapeDtypeStruct((B, C, N), jnp.float32),
    )(gsum, xs, x0, W1, W2, bias)


def kernel(x, x_0, edge_index, W1, W2, bias):
    B, C, N, _ = x.shape
    K = edge_index.shape[-1]
    deg = K + 1

    xs = jnp.swapaxes(x[..., 0], 1, 2)
    x2d = xs.reshape(B * N, C)
    idx1d = edge_index[0].reshape(B * N * K)

    gsum2d = _sc_gather_sum(x2d, idx1d, B, N, K, C)
    gsum = gsum2d.reshape(B, N, C)

    out3 = _tc_combine(gsum, xs, x_0, W1, W2, bias, B, N, C, deg)
    return out3[..., None]


# speedup vs baseline: 1.2751x; 1.0255x over previous
import functools
from math import log

import jax
import jax.numpy as jnp
from jax import lax
from jax.experimental import pallas as pl
from jax.experimental.pallas import tpu as pltpu
from jax.experimental.pallas import tpu_sc as plsc

_ALPHA = 0.1
_BETA = log(0.5 / 2 + 1.0)

_NC = 2
_NS = 16
_L = 16
_NW = _NC * _NS


def _sc_gather_sum(x2d, idx1d, B, N, K, C):
    total = B * N
    CH = 8
    assert N % CH == 0
    assert K == _L
    npw = ((total + _NW - 1) // _NW + CH - 1) // CH * CH
    ncw = npw // CH
    dg = C // _L
    NB = 6

    ncw_pad = (ncw + 7) // 8 * 8
    pad = _NW * npw * K - total * K
    idxp = jnp.concatenate([idx1d, jnp.zeros((pad,), jnp.int32)])
    idx3d = idxp.reshape(_NW, ncw, CH * K)
    idx3d = jnp.pad(idx3d, ((0, 0), (0, ncw_pad - ncw), (0, 0)))
    idx2d = idx3d.reshape(_NW * ncw_pad, CH * K)

    mesh = plsc.VectorSubcoreMesh(core_axis_name="c", subcore_axis_name="s")

    @functools.partial(
        pl.kernel,
        mesh=mesh,
        out_type=jax.ShapeDtypeStruct((total, C), jnp.float32),
        scratch_types=[
            pltpu.VMEM((ncw_pad, CH * K), jnp.int32),
            pltpu.VMEM((NB, CH * K, C), jnp.float32),
            pltpu.VMEM((2, CH, C), jnp.float32),
            pltpu.SemaphoreType.DMA,
            pltpu.SemaphoreType.DMA,
            pltpu.SemaphoreType.DMA,
            pltpu.SemaphoreType.DMA,
            pltpu.SemaphoreType.DMA,
            pltpu.SemaphoreType.DMA,
            pltpu.SemaphoreType.DMA,
            pltpu.SemaphoreType.DMA,
        ],
    )
    def k(x_hbm, idx_hbm, out_hbm, idx_v, rows_v, acc_v, *sems):
        wid = lax.axis_index("s") * _NC + lax.axis_index("c")
        base = wid * npw
        nch = jnp.maximum(jnp.minimum(total - base, npw), 0) // CH

        pltpu.sync_copy(idx_hbm.at[pl.ds(wid * ncw_pad, ncw_pad)], idx_v)

        def offs(r, carry):
            off = ((base + r * CH) // N) * N
            for q in range(CH):
                sl = pl.ds(q * _L, _L)
                idx_v[r, sl] = idx_v[r, sl] + off
            return carry

        lax.fori_loop(0, nch, offs, 0)

        def issue(c, buf):
            pltpu.async_copy(x_hbm.at[idx_v.at[c]], rows_v.at[buf], sems[buf])

        def wait(c, buf):
            pltpu.make_async_copy(
                x_hbm.at[idx_v.at[c]], rows_v.at[buf], sems[buf]
            ).wait()

        def reduce_store(c, buf, p):
            g0 = base + c * CH

            @pl.when(c >= 2)
            def _():
                pltpu.make_async_copy(
                    acc_v.at[p], out_hbm.at[pl.ds(g0, CH)], sems[6 + p]
                ).wait()

            def node(n, carry):
                for d in range(dg):
                    sl = pl.ds(d * _L, _L)
                    acc = rows_v[buf, n * K, sl]
                    for kk in range(1, K):
                        acc = acc + rows_v[buf, n * K + kk, sl]
                    acc_v[p, n, sl] = acc
                return carry

            lax.fori_loop(0, CH, node, 0)
            pltpu.async_copy(acc_v.at[p], out_hbm.at[pl.ds(g0, CH)], sems[6 + p])

        def body(i, carry):
            c0 = i * NB
            for b in range(NB):
                c = c0 + b

                @pl.when(c < nch)
                def _(c=c, b=b):
                    wait(c, b)
                    reduce_store(c, b, b % 2)

                @pl.when(c + NB < nch)
                def _(c=c, b=b):
                    issue(c + NB, b)

            return carry

        for b in range(NB):
            @pl.when(b < nch)
            def _(b=b):
                issue(b, b)
        lax.fori_loop(0, (nch + NB - 1) // NB, body, 0)

        for p in range(2):
            pltpu.make_async_copy(
                acc_v.at[p], out_hbm.at[pl.ds(base, CH)], sems[6 + p]
            ).wait()

    return k(x2d, idx2d)


def _tc_combine_body(g_ref, xs_ref, x0_ref, w1_ref, w2_ref, b_ref, out_ref, *, deg):
    inv = 1.0 / deg
    c_a = (1.0 - _ALPHA) * (1.0 - _BETA)
    c_0 = _ALPHA * (1.0 - _BETA)
    aggr = (g_ref[0] + xs_ref[0]) * inv
    x0 = x0_ref[0]
    r = aggr * c_a + jnp.dot(aggr, w1_ref[...], preferred_element_type=jnp.float32) * _BETA
    r = r + x0 * c_0 + jnp.dot(x0, w2_ref[...], preferred_element_type=jnp.float32) * _BETA
    r = r + b_ref[0, 0][None, :]
    r = jnp.maximum(r, 0.0)
    out_ref[0] = r.T


def _tc_combine(gsum, xs, x0, W1, W2, bias, B, N, C, deg):
    T = 2048
    grid = (B, (N + T - 1) // T)
    return pl.pallas_call(
        functools.partial(_tc_combine_body, deg=deg),
        grid=grid,
        in_specs=[
            pl.BlockSpec((1, T, C), lambda b, t: (b, t, 0)),
            pl.BlockSpec((1, T, C), lambda b, t: (b, t, 0)),
            pl.BlockSpec((1, T, C), lambda b, t: (b, t, 0)),
            pl.BlockSpec((C, C), lambda b, t: (0, 0)),
            pl.BlockSpec((C, C), lambda b, t: (0, 0)),
            pl.BlockSpec((1, 1, C), lambda b, t: (0, 0, 0)),
        ],
        out_specs=pl.BlockSpec((1, C, T), lambda b, t: (b, 0, t)),
        out_shape=jax.ShapeDtypeStruct((B, C, N), jnp.float32),
    )(gsum, xs, x0, W1, W2, bias)


def kernel(x, x_0, edge_index, W1, W2, bias):
    B, C, N, _ = x.shape
    K = edge_index.shape[-1]
    deg = K + 1

    xs = jnp.swapaxes(x[..., 0], 1, 2)
    x2d = xs.reshape(B * N, C)
    idx1d = edge_index[0].reshape(B * N * K)

    gsum2d = _sc_gather_sum(x2d, idx1d, B, N, K, C)
    gsum = gsum2d.reshape(B, N, C)

    out3 = _tc_combine(gsum, xs, x_0, W1, W2, bias, B, N, C, deg)
    return out3[..., None]


# speedup vs baseline: 1.2835x; 1.0066x over previous
import functools
from math import log

import jax
import jax.numpy as jnp
from jax import lax
from jax.experimental import pallas as pl
from jax.experimental.pallas import tpu as pltpu
from jax.experimental.pallas import tpu_sc as plsc

_ALPHA = 0.1
_BETA = log(0.5 / 2 + 1.0)

_NC = 2
_NS = 16
_L = 16
_NW = _NC * _NS


def _sc_gather_sum(x2d, idx1d, B, N, K, C):
    total = B * N
    CH = 8
    assert N % CH == 0
    assert K == _L
    npw = ((total + _NW - 1) // _NW + CH - 1) // CH * CH
    ncw = npw // CH
    dg = C // _L
    NB = 6

    ncw_pad = (ncw + 7) // 8 * 8
    pad = _NW * npw * K - total * K
    idxp = jnp.concatenate([idx1d, jnp.zeros((pad,), jnp.int32)])
    idx3d = idxp.reshape(_NW, ncw, CH * K)
    idx3d = jnp.pad(idx3d, ((0, 0), (0, ncw_pad - ncw), (0, 0)))
    idx2d = idx3d.reshape(_NW * ncw_pad, CH * K)

    mesh = plsc.VectorSubcoreMesh(core_axis_name="c", subcore_axis_name="s")

    @functools.partial(
        pl.kernel,
        mesh=mesh,
        out_type=jax.ShapeDtypeStruct((total, C), jnp.float32),
        scratch_types=[
            pltpu.VMEM((ncw_pad, CH * K), jnp.int32),
            pltpu.VMEM((NB, CH * K, C), jnp.float32),
            pltpu.VMEM((2, CH, C), jnp.float32),
            pltpu.SemaphoreType.DMA,
            pltpu.SemaphoreType.DMA,
            pltpu.SemaphoreType.DMA,
            pltpu.SemaphoreType.DMA,
            pltpu.SemaphoreType.DMA,
            pltpu.SemaphoreType.DMA,
            pltpu.SemaphoreType.DMA,
            pltpu.SemaphoreType.DMA,
        ],
    )
    def k(x_hbm, idx_hbm, out_hbm, idx_v, rows_v, acc_v, *sems):
        wid = lax.axis_index("s") * _NC + lax.axis_index("c")
        base = wid * npw
        nch = jnp.maximum(jnp.minimum(total - base, npw), 0) // CH

        pltpu.sync_copy(idx_hbm.at[pl.ds(wid * ncw_pad, ncw_pad)], idx_v)

        def offs(r, carry):
            off = ((base + r * CH) // N) * N
            for q in range(CH):
                sl = pl.ds(q * _L, _L)
                idx_v[r, sl] = idx_v[r, sl] + off
            return carry

        lax.fori_loop(0, nch, offs, 0)

        def issue(c, buf):
            pltpu.async_copy(x_hbm.at[idx_v.at[c]], rows_v.at[buf], sems[buf])

        def wait(c, buf):
            pltpu.make_async_copy(
                x_hbm.at[idx_v.at[c]], rows_v.at[buf], sems[buf]
            ).wait()

        def reduce_store(c, buf, p):
            g0 = base + c * CH

            @pl.when(c >= 2)
            def _():
                pltpu.make_async_copy(
                    acc_v.at[p], out_hbm.at[pl.ds(g0, CH)], sems[6 + p]
                ).wait()

            def node(n, carry):
                for d in range(dg):
                    sl = pl.ds(d * _L, _L)
                    acc = rows_v[buf, n * K, sl]
                    for kk in range(1, K):
                        acc = acc + rows_v[buf, n * K + kk, sl]
                    acc_v[p, n, sl] = acc
                return carry

            lax.fori_loop(0, CH, node, 0)
            pltpu.async_copy(acc_v.at[p], out_hbm.at[pl.ds(g0, CH)], sems[6 + p])

        def body(i, carry):
            c0 = i * NB
            for b in range(NB):
                c = c0 + b

                @pl.when(c < nch)
                def _(c=c, b=b):
                    wait(c, b)
                    reduce_store(c, b, b % 2)

                @pl.when(c + NB < nch)
                def _(c=c, b=b):
                    issue(c + NB, b)

            return carry

        for b in range(NB):
            @pl.when(b < nch)
            def _(b=b):
                issue(b, b)
        lax.fori_loop(0, (nch + NB - 1) // NB, body, 0)

        for p in range(2):
            pltpu.make_async_copy(
                acc_v.at[p], out_hbm.at[pl.ds(base, CH)], sems[6 + p]
            ).wait()

    return k(x2d, idx2d)


def _tc_combine_body(g_ref, xs_ref, x0_ref, w1_ref, w2_ref, b_ref, out_ref, *, deg):
    inv = 1.0 / deg
    c_a = (1.0 - _ALPHA) * (1.0 - _BETA)
    c_0 = _ALPHA * (1.0 - _BETA)
    aggr = (g_ref[0] + xs_ref[0]) * inv
    x0 = x0_ref[0]
    r = aggr * c_a + jnp.dot(aggr, w1_ref[...], preferred_element_type=jnp.float32) * _BETA
    r = r + x0 * c_0 + jnp.dot(x0, w2_ref[...], preferred_element_type=jnp.float32) * _BETA
    r = r + b_ref[0, 0][None, :]
    r = jnp.maximum(r, 0.0)
    out_ref[0] = r.T


def _tc_combine(gsum, xs, x0, W1, W2, bias, B, N, C, deg):
    T = 2560
    grid = (B, (N + T - 1) // T)
    return pl.pallas_call(
        functools.partial(_tc_combine_body, deg=deg),
        grid=grid,
        in_specs=[
            pl.BlockSpec((1, T, C), lambda b, t: (b, t, 0)),
            pl.BlockSpec((1, T, C), lambda b, t: (b, t, 0)),
            pl.BlockSpec((1, T, C), lambda b, t: (b, t, 0)),
            pl.BlockSpec((C, C), lambda b, t: (0, 0)),
            pl.BlockSpec((C, C), lambda b, t: (0, 0)),
            pl.BlockSpec((1, 1, C), lambda b, t: (0, 0, 0)),
        ],
        out_specs=pl.BlockSpec((1, C, T), lambda b, t: (b, 0, t)),
        out_shape=jax.ShapeDtypeStruct((B, C, N), jnp.float32),
    )(gsum, xs, x0, W1, W2, bias)


def kernel(x, x_0, edge_index, W1, W2, bias):
    B, C, N, _ = x.shape
    K = edge_index.shape[-1]
    deg = K + 1

    xs = jnp.swapaxes(x[..., 0], 1, 2)
    x2d = xs.reshape(B * N, C)
    idx1d = edge_index[0].reshape(B * N * K)

    gsum2d = _sc_gather_sum(x2d, idx1d, B, N, K, C)
    gsum = gsum2d.reshape(B, N, C)

    out3 = _tc_combine(gsum, xs, x_0, W1, W2, bias, B, N, C, deg)
    return out3[..., None]
